# Initial kernel scaffold; baseline (speedup 1.0000x reference)
import jax
import jax.numpy as jnp
from jax.experimental import pallas as pl


def kernel(x, ei, W1, b1, W2, b2, gamma, beta):
    raise NotImplementedError("write your pallas kernel here")


# speedup vs baseline: 2.1131x; 2.1131x over previous
import functools

import jax
import jax.numpy as jnp
from jax import lax
from jax.experimental import pallas as pl
from jax.experimental.pallas import tpu as pltpu
from jax.experimental.pallas import tpu_sc as plsc

N_NODES = 10000
N_EDGES = 320000
D = 128

NC = 2
NS = 16
NW = NC * NS
PER_W = N_EDGES // NW
CH = 80
ITERS = PER_W // CH

_LEAK = 0.01


def _leaky(v):
    return jnp.maximum(v, _LEAK * v)


def _pre_body(x_ref, w1_ref, b1_ref, a_ref, b_ref):
    xb = x_ref[...]
    w1 = w1_ref[...]
    wa = w1[:, :D]
    wb = w1[:, D:]
    dn = (((1,), (1,)), ((), ()))
    b_ref[...] = lax.dot_general(xb, wb, dn, preferred_element_type=jnp.float32)
    a_ref[...] = (
        lax.dot_general(xb, wa - wb, dn, preferred_element_type=jnp.float32)
        + b1_ref[...]
    )


def _pre(x, W1, b1r):
    blk = N_NODES // 10
    return pl.pallas_call(
        _pre_body,
        grid=(10,),
        in_specs=[
            pl.BlockSpec((blk, D), lambda i: (i, 0)),
            pl.BlockSpec((D, 2 * D), lambda i: (0, 0)),
            pl.BlockSpec((1, D), lambda i: (0, 0)),
        ],
        out_specs=[
            pl.BlockSpec((blk, D), lambda i: (i, 0)),
            pl.BlockSpec((blk, D), lambda i: (i, 0)),
        ],
        out_shape=[
            jax.ShapeDtypeStruct((N_NODES, D), jnp.float32),
            jax.ShapeDtypeStruct((N_NODES, D), jnp.float32),
        ],
    )(x, W1, b1r)


def _sc_msg_body(a_hbm, b_hbm, dst_hbm, src_hbm, msg_out,
                 idxd, idxs, arows, brows, sem):
    cid = lax.axis_index("c")
    sid = lax.axis_index("s")
    wid = cid * NS + sid

    def chunk(it, _):
        base = wid * PER_W + it * CH
        pltpu.sync_copy(dst_hbm.at[pl.ds(base, CH)], idxd)
        pltpu.sync_copy(src_hbm.at[pl.ds(base, CH)], idxs)
        cp1 = pltpu.async_copy(a_hbm.at[idxd], arows, sem)
        cp2 = pltpu.async_copy(b_hbm.at[idxs], brows, sem)
        cp1.wait()
        cp2.wait()

        def row(r, _):
            for c in range(D // 16):
                s = pl.ds(c * 16, 16)
                m = arows[r, s] + brows[r, s]
                arows[r, s] = jnp.maximum(m, _LEAK * m)
            return 0

        lax.fori_loop(0, CH, row, 0)
        pltpu.sync_copy(arows, msg_out.at[pl.ds(base, CH)])
        return 0

    lax.fori_loop(0, ITERS, chunk, 0)


_sc_msg = functools.partial(
    pl.kernel,
    out_type=jax.ShapeDtypeStruct((N_EDGES, D), jnp.float32),
    mesh=plsc.VectorSubcoreMesh(core_axis_name="c", subcore_axis_name="s",
                                num_cores=NC, num_subcores=NS),
    scratch_types=[
        pltpu.VMEM((CH,), jnp.int32),
        pltpu.VMEM((CH,), jnp.int32),
        pltpu.VMEM((CH, D), jnp.float32),
        pltpu.VMEM((CH, D), jnp.float32),
        pltpu.SemaphoreType.DMA,
    ],
)(_sc_msg_body)


def _post_body(acc_ref, cnt_ref, x_ref, w2_ref, b2_ref, g_ref, be_ref, o_ref):
    acc = acc_ref[...]
    cnt = cnt_ref[...]
    p = acc / jnp.maximum(cnt, 1.0)
    dn = (((1,), (1,)), ((), ()))
    g = lax.dot_general(p, w2_ref[...], dn, preferred_element_type=jnp.float32)
    g = g + b2_ref[...] * (cnt > 0).astype(jnp.float32)
    mu = jnp.mean(g, axis=1, keepdims=True)
    var = jnp.mean((g - mu) ** 2, axis=1, keepdims=True)
    hn = (g - mu) / jnp.sqrt(var + 1e-5) * g_ref[...] + be_ref[...]
    hn = hn + x_ref[...]
    o_ref[...] = _leaky(hn)


def _post(acc, cnt, x, W2, b2r, gr, ber):
    blk = 400
    grid = N_NODES // blk
    return pl.pallas_call(
        _post_body,
        grid=(grid,),
        in_specs=[
            pl.BlockSpec((blk, D), lambda i: (i, 0)),
            pl.BlockSpec((blk, 1), lambda i: (i, 0)),
            pl.BlockSpec((blk, D), lambda i: (i, 0)),
            pl.BlockSpec((D, D), lambda i: (0, 0)),
            pl.BlockSpec((1, D), lambda i: (0, 0)),
            pl.BlockSpec((1, D), lambda i: (0, 0)),
            pl.BlockSpec((1, D), lambda i: (0, 0)),
        ],
        out_specs=pl.BlockSpec((blk, D), lambda i: (i, 0)),
        out_shape=jax.ShapeDtypeStruct((N_NODES, D), jnp.float32),
    )(acc, cnt, x, W2, b2r, gr, ber)


def kernel(x, ei, W1, b1, W2, b2, gamma, beta):
    ei32 = ei.astype(jnp.int32)
    src = ei32[0]
    dst = ei32[1]

    a_tab, b_tab = _pre(x, W1, b1.reshape(1, D))

    msg = _sc_msg(a_tab, b_tab, dst, src)

    acc = jax.ops.segment_sum(msg, dst, num_segments=N_NODES)
    cnt = jax.ops.segment_sum(jnp.ones((N_EDGES, 1), jnp.float32), dst,
                              num_segments=N_NODES)

    return _post(acc, cnt, x, W2, b2.reshape(1, D),
                 gamma.reshape(1, D), beta.reshape(1, D))


# speedup vs baseline: 2.1140x; 1.0004x over previous
import functools

import jax
import jax.numpy as jnp
from jax import lax
from jax.experimental import pallas as pl
from jax.experimental.pallas import tpu as pltpu
from jax.experimental.pallas import tpu_sc as plsc

N_NODES = 10000
N_EDGES = 320000
D = 128

NC = 2
NS = 16
NW = NC * NS
PER_W = N_EDGES // NW
CH = 80
ITERS = PER_W // CH

_LEAK = 0.01


def _leaky(v):
    return jnp.maximum(v, _LEAK * v)


def _pre_body(x_ref, w1_ref, b1_ref, a_ref, b_ref):
    xb = x_ref[...]
    w1 = w1_ref[...]
    wa = w1[:, :D]
    wb = w1[:, D:]
    dn = (((1,), (1,)), ((), ()))
    b_ref[...] = lax.dot_general(xb, wb, dn, preferred_element_type=jnp.float32)
    a_ref[...] = (
        lax.dot_general(xb, wa - wb, dn, preferred_element_type=jnp.float32)
        + b1_ref[...]
    )


def _pre(x, W1, b1r):
    blk = N_NODES // 10
    return pl.pallas_call(
        _pre_body,
        grid=(10,),
        in_specs=[
            pl.BlockSpec((blk, D), lambda i: (i, 0)),
            pl.BlockSpec((D, 2 * D), lambda i: (0, 0)),
            pl.BlockSpec((1, D), lambda i: (0, 0)),
        ],
        out_specs=[
            pl.BlockSpec((blk, D), lambda i: (i, 0)),
            pl.BlockSpec((blk, D), lambda i: (i, 0)),
        ],
        out_shape=[
            jax.ShapeDtypeStruct((N_NODES, D), jnp.float32),
            jax.ShapeDtypeStruct((N_NODES, D), jnp.float32),
        ],
    )(x, W1, b1r)


def _sc_msg_body(a_hbm, b_hbm, dst_hbm, src_hbm, msg_out,
                 idx0, ar0, br0, idx1, ar1, br1, sem0, sem1):
    cid = lax.axis_index("c")
    sid = lax.axis_index("s")
    wid = cid * NS + sid
    ebase = wid * PER_W

    slots = ((idx0, ar0, br0, sem0),
             (idx1, ar1, br1, sem1))

    def prefetch(k, slot):
        idx, ar, br, sem = slot
        base = ebase + k * CH
        pltpu.sync_copy(dst_hbm.at[pl.ds(base, CH)], idx.at[pl.ds(0, CH)])
        pltpu.sync_copy(src_hbm.at[pl.ds(base, CH)], idx.at[pl.ds(CH, CH)])
        pltpu.async_copy(a_hbm.at[idx.at[pl.ds(0, CH)]], ar, sem)
        pltpu.async_copy(b_hbm.at[idx.at[pl.ds(CH, CH)]], br, sem)

    def process(k, slot):
        idx, ar, br, sem = slot
        pltpu.make_async_copy(a_hbm.at[pl.ds(0, CH)], ar, sem).wait()
        pltpu.make_async_copy(b_hbm.at[pl.ds(0, CH)], br, sem).wait()

        def row(r, _):
            for c in range(D // 16):
                s = pl.ds(c * 16, 16)
                m = ar[r, s] + br[r, s]
                ar[r, s] = jnp.maximum(m, _LEAK * m)
            return 0

        lax.fori_loop(0, CH, row, 0)
        pltpu.sync_copy(ar, msg_out.at[pl.ds(ebase + k * CH, CH)])

    prefetch(0, slots[0])

    def pair(i, _):
        prefetch(2 * i + 1, slots[1])
        process(2 * i, slots[0])
        prefetch(2 * i + 2, slots[0])
        process(2 * i + 1, slots[1])
        return 0

    lax.fori_loop(0, (ITERS - 1) // 2, pair, 0)
    process(ITERS - 1, slots[0])


_sc_msg = functools.partial(
    pl.kernel,
    out_type=jax.ShapeDtypeStruct((N_EDGES, D), jnp.float32),
    mesh=plsc.VectorSubcoreMesh(core_axis_name="c", subcore_axis_name="s",
                                num_cores=NC, num_subcores=NS),
    scratch_types=[
        pltpu.VMEM((2 * CH,), jnp.int32),
        pltpu.VMEM((CH, D), jnp.float32),
        pltpu.VMEM((CH, D), jnp.float32),
        pltpu.VMEM((2 * CH,), jnp.int32),
        pltpu.VMEM((CH, D), jnp.float32),
        pltpu.VMEM((CH, D), jnp.float32),
        pltpu.SemaphoreType.DMA,
        pltpu.SemaphoreType.DMA,
    ],
)(_sc_msg_body)


def _post_body(acc_ref, cnt_ref, x_ref, w2_ref, b2_ref, g_ref, be_ref, o_ref):
    acc = acc_ref[...]
    cnt = cnt_ref[...]
    p = acc / jnp.maximum(cnt, 1.0)
    dn = (((1,), (1,)), ((), ()))
    g = lax.dot_general(p, w2_ref[...], dn, preferred_element_type=jnp.float32)
    g = g + b2_ref[...] * (cnt > 0).astype(jnp.float32)
    mu = jnp.mean(g, axis=1, keepdims=True)
    var = jnp.mean((g - mu) ** 2, axis=1, keepdims=True)
    hn = (g - mu) / jnp.sqrt(var + 1e-5) * g_ref[...] + be_ref[...]
    hn = hn + x_ref[...]
    o_ref[...] = _leaky(hn)


def _post(acc, cnt, x, W2, b2r, gr, ber):
    blk = 400
    grid = N_NODES // blk
    return pl.pallas_call(
        _post_body,
        grid=(grid,),
        in_specs=[
            pl.BlockSpec((blk, D), lambda i: (i, 0)),
            pl.BlockSpec((blk, 1), lambda i: (i, 0)),
            pl.BlockSpec((blk, D), lambda i: (i, 0)),
            pl.BlockSpec((D, D), lambda i: (0, 0)),
            pl.BlockSpec((1, D), lambda i: (0, 0)),
            pl.BlockSpec((1, D), lambda i: (0, 0)),
            pl.BlockSpec((1, D), lambda i: (0, 0)),
        ],
        out_specs=pl.BlockSpec((blk, D), lambda i: (i, 0)),
        out_shape=jax.ShapeDtypeStruct((N_NODES, D), jnp.float32),
    )(acc, cnt, x, W2, b2r, gr, ber)


def kernel(x, ei, W1, b1, W2, b2, gamma, beta):
    ei32 = ei.astype(jnp.int32)
    src = ei32[0]
    dst = ei32[1]

    a_tab, b_tab = _pre(x, W1, b1.reshape(1, D))

    msg = _sc_msg(a_tab, b_tab, dst, src)

    acc = jax.ops.segment_sum(msg, dst, num_segments=N_NODES)
    cnt = jax.ops.segment_sum(jnp.ones((N_EDGES, 1), jnp.float32), dst,
                              num_segments=N_NODES)

    return _post(acc, cnt, x, W2, b2.reshape(1, D),
                 gamma.reshape(1, D), beta.reshape(1, D))


# speedup vs baseline: 2.1146x; 1.0003x over previous
import functools

import jax
import jax.numpy as jnp
from jax import lax
from jax.experimental import pallas as pl
from jax.experimental.pallas import tpu as pltpu
from jax.experimental.pallas import tpu_sc as plsc

N_NODES = 10000
N_EDGES = 320000
D = 128

NC = 2
NS = 16
NW = NC * NS
PER_W = N_EDGES // NW
CH = 80
ITERS = PER_W // CH

_LEAK = 0.01


def _leaky(v):
    return jnp.maximum(v, _LEAK * v)


def _pre_body(x_ref, w1_ref, b1_ref, a_ref, b_ref):
    xb = x_ref[...]
    w1 = w1_ref[...]
    wa = w1[:, :D]
    wb = w1[:, D:]
    dn = (((1,), (1,)), ((), ()))
    b_ref[...] = lax.dot_general(xb, wb, dn, preferred_element_type=jnp.float32)
    a_ref[...] = (
        lax.dot_general(xb, wa - wb, dn, preferred_element_type=jnp.float32)
        + b1_ref[...]
    )


def _pre(x, W1, b1r):
    blk = N_NODES // 10
    return pl.pallas_call(
        _pre_body,
        grid=(10,),
        in_specs=[
            pl.BlockSpec((blk, D), lambda i: (i, 0)),
            pl.BlockSpec((D, 2 * D), lambda i: (0, 0)),
            pl.BlockSpec((1, D), lambda i: (0, 0)),
        ],
        out_specs=[
            pl.BlockSpec((blk, D), lambda i: (i, 0)),
            pl.BlockSpec((blk, D), lambda i: (i, 0)),
        ],
        out_shape=[
            jax.ShapeDtypeStruct((N_NODES, D), jnp.float32),
            jax.ShapeDtypeStruct((N_NODES, D), jnp.float32),
        ],
    )(x, W1, b1r)


def _sc_msg_body(a_hbm, b_hbm, dst_hbm, src_hbm, msg_out,
                 idx0, ar0, br0, idx1, ar1, br1, sem0, sem1):
    cid = lax.axis_index("c")
    sid = lax.axis_index("s")
    wid = cid * NS + sid
    ebase = wid * PER_W

    slots = ((idx0, ar0, br0, sem0),
             (idx1, ar1, br1, sem1))

    def prefetch(k, slot):
        idx, ar, br, sem = slot
        base = ebase + k * CH
        pltpu.sync_copy(dst_hbm.at[pl.ds(base, CH)], idx.at[pl.ds(0, CH)])
        pltpu.sync_copy(src_hbm.at[pl.ds(base, CH)], idx.at[pl.ds(CH, CH)])
        pltpu.async_copy(a_hbm.at[idx.at[pl.ds(0, CH)]], ar, sem)
        pltpu.async_copy(b_hbm.at[idx.at[pl.ds(CH, CH)]], br, sem)

    def process(k, slot):
        idx, ar, br, sem = slot
        pltpu.make_async_copy(a_hbm.at[pl.ds(0, CH)], ar, sem).wait()
        pltpu.make_async_copy(b_hbm.at[pl.ds(0, CH)], br, sem).wait()

        @plsc.parallel_loop(0, CH, unroll=4)
        def _(r):
            for c in range(D // 16):
                s = pl.ds(c * 16, 16)
                m = ar[r, s] + br[r, s]
                ar[r, s] = jnp.maximum(m, _LEAK * m)
        pltpu.sync_copy(ar, msg_out.at[pl.ds(ebase + k * CH, CH)])

    prefetch(0, slots[0])

    def pair(i, _):
        prefetch(2 * i + 1, slots[1])
        process(2 * i, slots[0])
        prefetch(2 * i + 2, slots[0])
        process(2 * i + 1, slots[1])
        return 0

    lax.fori_loop(0, (ITERS - 1) // 2, pair, 0)
    process(ITERS - 1, slots[0])


_sc_msg = functools.partial(
    pl.kernel,
    out_type=jax.ShapeDtypeStruct((N_EDGES, D), jnp.float32),
    mesh=plsc.VectorSubcoreMesh(core_axis_name="c", subcore_axis_name="s",
                                num_cores=NC, num_subcores=NS),
    scratch_types=[
        pltpu.VMEM((2 * CH,), jnp.int32),
        pltpu.VMEM((CH, D), jnp.float32),
        pltpu.VMEM((CH, D), jnp.float32),
        pltpu.VMEM((2 * CH,), jnp.int32),
        pltpu.VMEM((CH, D), jnp.float32),
        pltpu.VMEM((CH, D), jnp.float32),
        pltpu.SemaphoreType.DMA,
        pltpu.SemaphoreType.DMA,
    ],
)(_sc_msg_body)


def _post_body(acc_ref, cnt_ref, x_ref, w2_ref, b2_ref, g_ref, be_ref, o_ref):
    acc = acc_ref[...]
    cnt = cnt_ref[...]
    p = acc / jnp.maximum(cnt, 1.0)
    dn = (((1,), (1,)), ((), ()))
    g = lax.dot_general(p, w2_ref[...], dn, preferred_element_type=jnp.float32)
    g = g + b2_ref[...] * (cnt > 0).astype(jnp.float32)
    mu = jnp.mean(g, axis=1, keepdims=True)
    var = jnp.mean((g - mu) ** 2, axis=1, keepdims=True)
    hn = (g - mu) / jnp.sqrt(var + 1e-5) * g_ref[...] + be_ref[...]
    hn = hn + x_ref[...]
    o_ref[...] = _leaky(hn)


def _post(acc, cnt, x, W2, b2r, gr, ber):
    blk = 400
    grid = N_NODES // blk
    return pl.pallas_call(
        _post_body,
        grid=(grid,),
        in_specs=[
            pl.BlockSpec((blk, D), lambda i: (i, 0)),
            pl.BlockSpec((blk, 1), lambda i: (i, 0)),
            pl.BlockSpec((blk, D), lambda i: (i, 0)),
            pl.BlockSpec((D, D), lambda i: (0, 0)),
            pl.BlockSpec((1, D), lambda i: (0, 0)),
            pl.BlockSpec((1, D), lambda i: (0, 0)),
            pl.BlockSpec((1, D), lambda i: (0, 0)),
        ],
        out_specs=pl.BlockSpec((blk, D), lambda i: (i, 0)),
        out_shape=jax.ShapeDtypeStruct((N_NODES, D), jnp.float32),
    )(acc, cnt, x, W2, b2r, gr, ber)


def kernel(x, ei, W1, b1, W2, b2, gamma, beta):
    ei32 = ei.astype(jnp.int32)
    src = ei32[0]
    dst = ei32[1]

    a_tab, b_tab = _pre(x, W1, b1.reshape(1, D))

    msg = _sc_msg(a_tab, b_tab, dst, src)

    acc = jax.ops.segment_sum(msg, dst, num_segments=N_NODES)
    cnt = jax.ops.segment_sum(jnp.ones((N_EDGES, 1), jnp.float32), dst,
                              num_segments=N_NODES)

    return _post(acc, cnt, x, W2, b2.reshape(1, D),
                 gamma.reshape(1, D), beta.reshape(1, D))
